# baseline (speedup 1.0000x reference)
import functools

import jax
import jax.numpy as jnp
from jax import lax
from jax.experimental import pallas as pl
from jax.experimental.pallas import tpu as pltpu
from jax.experimental.pallas import tpu_sc as plsc

NC, NS, L = 2, 16, 16
NW = NC * NS
BN1_EPS = 1e-5
BN2_EPS = 1e-5


def _project(atom, w0t, w1t):
    N, A = atom.shape
    D = w0t.shape[1]
    BN = 2000

    def body(x_ref, w0_ref, w1_ref, p0_ref, p1_ref):
        x = x_ref[...]
        p0_ref[...] = jnp.dot(x, w0_ref[...],
                              preferred_element_type=jnp.float32
                              ).astype(jnp.bfloat16)
        p1_ref[...] = jnp.dot(x, w1_ref[...],
                              preferred_element_type=jnp.float32
                              ).astype(jnp.bfloat16)

    return pl.pallas_call(
        body,
        grid=(N // BN,),
        in_specs=[
            pl.BlockSpec((BN, A), lambda i: (i, 0)),
            pl.BlockSpec((A, D), lambda i: (0, 0)),
            pl.BlockSpec((A, D), lambda i: (0, 0)),
        ],
        out_specs=[
            pl.BlockSpec((BN, D), lambda i: (i, 0)),
            pl.BlockSpec((BN, D), lambda i: (i, 0)),
        ],
        out_shape=[jax.ShapeDtypeStruct((N, D), jnp.bfloat16)] * 2,
    )(atom, w0t, w1t)


def _sc_gather_combine(p0, p1, idx0, idx1):
    E = idx0.shape[0]
    Dw = p0.shape[1]
    D = 2 * Dw
    per_w = E // NW
    CH = 48
    n_chunks = per_w // CH
    mesh = plsc.VectorSubcoreMesh(core_axis_name="c", subcore_axis_name="s")

    @functools.partial(
        pl.kernel,
        out_type=jax.ShapeDtypeStruct((E, D), jnp.float32),
        mesh=mesh,
        scratch_types=[
            pltpu.VMEM((per_w,), jnp.int32),
            pltpu.VMEM((per_w,), jnp.int32),
            pltpu.VMEM((CH, Dw), jnp.int32),
            pltpu.VMEM((CH, Dw), jnp.int32),
            pltpu.VMEM((CH, D), jnp.float32),
            pltpu.VMEM((CH, Dw), jnp.int32),
            pltpu.VMEM((CH, Dw), jnp.int32),
            pltpu.VMEM((CH, D), jnp.float32),
            pltpu.SemaphoreType.DMA,
            pltpu.SemaphoreType.DMA,
            pltpu.SemaphoreType.DMA,
            pltpu.SemaphoreType.DMA,
            pltpu.SemaphoreType.DMA,
            pltpu.SemaphoreType.DMA,
        ],
    )
    def k(p0_hbm, p1_hbm, i0_hbm, i1_hbm, out_hbm, i0_v, i1_v,
          buf_a0, buf_b0, buf_o0, buf_a1, buf_b1, buf_o1,
          sem_a0, sem_b0, sem_a1, sem_b1, sem_s0, sem_s1):
        wid = lax.axis_index("s") * NC + lax.axis_index("c")
        base = wid * per_w
        pltpu.sync_copy(i0_hbm.at[pl.ds(base, per_w)], i0_v)
        pltpu.sync_copy(i1_hbm.at[pl.ds(base, per_w)], i1_v)

        bufs = ((buf_a0, buf_b0, buf_o0, sem_a0, sem_b0, sem_s0),
                (buf_a1, buf_b1, buf_o1, sem_a1, sem_b1, sem_s1))

        def issue(slot, ci):
            buf_a, buf_b, _, sem_a, sem_b, _ = bufs[slot]
            off = ci * CH
            pltpu.async_copy(p0_hbm.at[i0_v.at[pl.ds(off, CH)]], buf_a, sem_a)
            pltpu.async_copy(p1_hbm.at[i1_v.at[pl.ds(off, CH)]], buf_b, sem_b)

        def wait_gathers(slot, ci):
            buf_a, buf_b, _, sem_a, sem_b, _ = bufs[slot]
            off = ci * CH
            pltpu.make_async_copy(
                p0_hbm.at[i0_v.at[pl.ds(off, CH)]], buf_a, sem_a).wait()
            pltpu.make_async_copy(
                p1_hbm.at[i1_v.at[pl.ds(off, CH)]], buf_b, sem_b).wait()

        def combine(slot):
            buf_a, buf_b, buf_o, _, _, _ = bufs[slot]
            bc = lax.bitcast_convert_type

            def row(e, c2):
                for j in range(Dw // L):
                    sl = pl.ds(j * L, L)
                    a = buf_a[e, sl]
                    b = buf_b[e, sl]
                    lo = (bc(lax.shift_left(a, 16), jnp.float32)
                          + bc(lax.shift_left(b, 16), jnp.float32))
                    hi = (bc(a & jnp.int32(-65536), jnp.float32)
                          + bc(b & jnp.int32(-65536), jnp.float32))
                    buf_o[e, pl.ds(j * L, L)] = lo
                    buf_o[e, pl.ds(Dw + j * L, L)] = hi
                return c2

            lax.fori_loop(0, CH, row, 0, unroll=False)

        def issue_store(slot, ci):
            _, _, buf_o, _, _, sem_s = bufs[slot]
            pltpu.async_copy(buf_o, out_hbm.at[pl.ds(base + ci * CH, CH)],
                             sem_s)

        def wait_store(slot, ci):
            _, _, buf_o, _, _, sem_s = bufs[slot]
            pltpu.make_async_copy(
                buf_o, out_hbm.at[pl.ds(base + ci * CH, CH)], sem_s).wait()

        n_pairs = (n_chunks - 1) // 2
        issue(0, 0)
        issue(1, 1)

        def pair(kk, carry):
            c0 = 2 * kk
            wait_gathers(0, c0)
            combine(0)
            issue_store(0, c0)
            wait_gathers(1, c0 + 1)
            combine(1)
            issue_store(1, c0 + 1)

            @pl.when(kk + 1 < n_pairs)
            def _():
                wait_store(0, c0)
                issue(0, c0 + 2)
                wait_store(1, c0 + 1)
                issue(1, c0 + 3)

            return carry

        lax.fori_loop(0, n_pairs, pair, 0, unroll=False)
        last = n_chunks - 1
        wait_store(0, last - 2)
        wait_store(1, last - 1)
        issue(0, last)
        wait_gathers(0, last)
        combine(0)
        pltpu.sync_copy(buf_o0, out_hbm.at[pl.ds(base + last * CH, CH)])

    return k(p0, p1, idx0, idx1)


def _edge_stats(g, nbr, w2t_s):
    E, Bf = nbr.shape
    D = w2t_s.shape[1]
    BE = 2000

    def body(g_ref, nbr_ref, w2_ref, sum_ref, sq_ref):
        i = pl.program_id(0)
        gated = g_ref[...] + jnp.dot(
            nbr_ref[...], w2_ref[...], preferred_element_type=jnp.float32)
        s = jnp.sum(gated, axis=0, keepdims=True)
        q = jnp.sum(gated * gated, axis=0, keepdims=True)

        @pl.when(i == 0)
        def _():
            sum_ref[...] = s
            sq_ref[...] = q

        @pl.when(i != 0)
        def _():
            sum_ref[...] += s
            sq_ref[...] += q

    return pl.pallas_call(
        body,
        grid=(E // BE,),
        in_specs=[
            pl.BlockSpec((BE, D), lambda i: (i, 0)),
            pl.BlockSpec((BE, Bf), lambda i: (i, 0)),
            pl.BlockSpec((Bf, D), lambda i: (0, 0)),
        ],
        out_specs=[
            pl.BlockSpec((1, D), lambda i: (0, 0)),
            pl.BlockSpec((1, D), lambda i: (0, 0)),
        ],
        out_shape=[jax.ShapeDtypeStruct((1, D), jnp.float32)] * 2,
    )(g, nbr, w2t_s)


def _edge_messages(g, nbr, w2t_s, gsum, gsq, gamma1_s, beta1_s):
    E, Bf = nbr.shape
    D = w2t_s.shape[1]
    Dw = D // 2
    A = D // 2
    H = A // 2
    BE = 2000
    inv_e = 1.0 / E

    def body(g_ref, nbr_ref, w2_ref, sum_ref, sq_ref, gam_ref, bet_ref, o_ref):
        gated = g_ref[...] + jnp.dot(
            nbr_ref[...], w2_ref[...], preferred_element_type=jnp.float32)
        mean = sum_ref[...] * inv_e
        var = sq_ref[...] * inv_e - mean * mean
        scale = lax.rsqrt(var + BN1_EPS) * gam_ref[...]
        shift = bet_ref[...] - mean * scale
        xh = gated * scale + shift
        f = xh[:, :A]
        c = xh[:, A:]
        sig = 1.0 / (1.0 + jnp.exp(-f))
        sp = jnp.maximum(c, 0.0) + jnp.log(1.0 + jnp.exp(-jnp.abs(c)))
        msg = sig * sp
        o_ref[0, ...] = msg[:, :H]
        o_ref[1, ...] = msg[:, H:]

    return pl.pallas_call(
        body,
        grid=(E // BE,),
        in_specs=[
            pl.BlockSpec((BE, D), lambda i: (i, 0)),
            pl.BlockSpec((BE, Bf), lambda i: (i, 0)),
            pl.BlockSpec((Bf, D), lambda i: (0, 0)),
            pl.BlockSpec((1, D), lambda i: (0, 0)),
            pl.BlockSpec((1, D), lambda i: (0, 0)),
            pl.BlockSpec((1, D), lambda i: (0, 0)),
            pl.BlockSpec((1, D), lambda i: (0, 0)),
        ],
        out_specs=pl.BlockSpec((2, BE, H), lambda i: (0, i, 0)),
        out_shape=jax.ShapeDtypeStruct((2, E, H), jnp.float32),
    )(g, nbr, w2t_s, gsum, gsq, gamma1_s, beta1_s)


def _sc_scatter_add(msg2, idx_tiled, n_nodes, zeros_init):
    _, E, half = msg2.shape
    per_t = E // NS
    CH = 80
    n_chunks = per_t // CH
    out_writers = 10
    rows_out = n_nodes // out_writers
    mesh = plsc.VectorSubcoreMesh(core_axis_name="c", subcore_axis_name="s")

    @functools.partial(
        pl.kernel,
        out_type=jax.ShapeDtypeStruct((NC, n_nodes, half), jnp.float32),
        mesh=mesh,
        scratch_types=[
            pltpu.VMEM((n_chunks, CH), jnp.int32),
            pltpu.VMEM((CH, half), jnp.float32),
            pltpu.VMEM((CH, half), jnp.float32),
            pltpu.VMEM_SHARED((n_nodes, half), jnp.float32),
            pltpu.SemaphoreType.DMA,
            pltpu.SemaphoreType.DMA,
        ],
    )
    def k(msg_hbm, idx_hbm, zero_hbm, out_hbm, idx_v, buf0, buf1, acc_sh,
          sem0, sem1):
        cid = lax.axis_index("c")
        sid = lax.axis_index("s")
        tbase = sid * per_t
        pltpu.sync_copy(idx_hbm.at[sid], idx_v)

        @pl.when(sid == 0)
        def _():
            pltpu.sync_copy(zero_hbm, acc_sh)

        plsc.subcore_barrier()

        bufs = ((buf0, sem0), (buf1, sem1))

        def issue(slot, j):
            buf, sem = bufs[slot]
            pltpu.async_copy(
                msg_hbm.at[cid, pl.ds(tbase + j * CH, CH)], buf, sem)

        def scat(slot, j):
            buf, sem = bufs[slot]
            pltpu.make_async_copy(
                msg_hbm.at[cid, pl.ds(tbase + j * CH, CH)], buf, sem).wait()
            pltpu.sync_copy(buf, acc_sh.at[idx_v.at[j]], add=True)

        n_pairs = (n_chunks - 1) // 2
        issue(0, 0)
        issue(1, 1)

        def pair(kk, carry):
            j0 = 2 * kk
            scat(0, j0)
            issue(0, j0 + 2)
            scat(1, j0 + 1)

            @pl.when(kk + 1 < n_pairs)
            def _():
                issue(1, j0 + 3)

            return carry

        lax.fori_loop(0, n_pairs, pair, 0, unroll=False)
        scat(0, n_chunks - 1)
        plsc.subcore_barrier()

        @pl.when(sid < out_writers)
        def _():
            pltpu.sync_copy(
                acc_sh.at[pl.ds(sid * rows_out, rows_out)],
                out_hbm.at[cid, pl.ds(sid * rows_out, rows_out)])

    return k(msg2, idx_tiled, zeros_init)


def _finalize(nbr_sumed, atom, gamma2, beta2):
    N, A = atom.shape
    inv_n = 1.0 / N

    def body(s_ref, a_ref, g_ref, b_ref, o_ref):
        x = jnp.concatenate((s_ref[0, ...], s_ref[1, ...]), axis=1)
        mean = jnp.sum(x, axis=0, keepdims=True) * inv_n
        d = x - mean
        var = jnp.sum(d * d, axis=0, keepdims=True) * inv_n
        xh = d * lax.rsqrt(var + BN2_EPS) * g_ref[...] + b_ref[...]
        y = a_ref[...] + xh
        o_ref[...] = jnp.maximum(y, 0.0) + jnp.log(1.0 + jnp.exp(-jnp.abs(y)))

    return pl.pallas_call(
        body,
        out_shape=jax.ShapeDtypeStruct((N, A), jnp.float32),
    )(nbr_sumed, atom, gamma2, beta2)


def kernel(atom_in_fea, nbr_fea, nbr_fea_idx, W, b, gamma1, beta1, gamma2,
           beta2):
    N, A = atom_in_fea.shape
    E = nbr_fea_idx.shape[0]
    D = 2 * A

    w0t = W[:, :A].T
    w1t = W[:, A:2 * A].T
    w2t = W[:, 2 * A:].T
    idx0 = nbr_fea_idx[:, 0].astype(jnp.int32)
    idx1 = nbr_fea_idx[:, 1].astype(jnp.int32)

    e_pad = -(-E // (NW * 48)) * (NW * 48)
    pad = e_pad - E
    idx0_p = jnp.concatenate([idx0, jnp.zeros((pad,), jnp.int32)])
    idx1_p = jnp.concatenate([idx1, jnp.zeros((pad,), jnp.int32)])

    perm = jnp.arange(D).reshape(2, A).T.reshape(D)
    p0, p1 = _project(atom_in_fea, w0t[:, perm], w1t[:, perm])
    p0i = lax.bitcast_convert_type(p0.reshape(N, A, 2), jnp.int32)
    p1i = lax.bitcast_convert_type(p1.reshape(N, A, 2), jnp.int32)
    g = _sc_gather_combine(p0i, p1i, idx0_p, idx1_p)

    gsum, gsq = _edge_stats(g, nbr_fea, w2t)
    msg = _edge_messages(g, nbr_fea, w2t, gsum, gsq,
                         gamma1.reshape(1, D), beta1.reshape(1, D))

    idx_tiled = idx0.reshape(NS, (E // NS) // 80, 80)
    zeros_init = jnp.zeros((N, A // NC), jnp.float32)
    nbr_sumed2 = _sc_scatter_add(msg, idx_tiled, N, zeros_init)

    return _finalize(nbr_sumed2, atom_in_fea,
                     gamma2.reshape(1, A), beta2.reshape(1, A))


# speedup vs baseline: 1.2439x; 1.2439x over previous
import functools

import jax
import jax.numpy as jnp
from jax import lax
from jax.experimental import pallas as pl
from jax.experimental.pallas import tpu as pltpu
from jax.experimental.pallas import tpu_sc as plsc

NC, NS, L = 2, 16, 16
NW = NC * NS
BN1_EPS = 1e-5
BN2_EPS = 1e-5


def _project(atom, w0t, w1t):
    N, A = atom.shape
    D = w0t.shape[1]
    BN = 2000

    def body(x_ref, w0_ref, w1_ref, p0_ref, p1_ref):
        x = x_ref[...]
        p0_ref[...] = jnp.dot(x, w0_ref[...],
                              preferred_element_type=jnp.float32
                              ).astype(jnp.bfloat16)
        p1_ref[...] = jnp.dot(x, w1_ref[...],
                              preferred_element_type=jnp.float32
                              ).astype(jnp.bfloat16)

    return pl.pallas_call(
        body,
        grid=(N // BN,),
        in_specs=[
            pl.BlockSpec((BN, A), lambda i: (i, 0)),
            pl.BlockSpec((A, D), lambda i: (0, 0)),
            pl.BlockSpec((A, D), lambda i: (0, 0)),
        ],
        out_specs=[
            pl.BlockSpec((BN, D), lambda i: (i, 0)),
            pl.BlockSpec((BN, D), lambda i: (i, 0)),
        ],
        out_shape=[jax.ShapeDtypeStruct((N, D), jnp.bfloat16)] * 2,
    )(atom, w0t, w1t)


def _sc_gather_combine(p0, p1, idx0, idx1):
    E = idx0.shape[0]
    Dw = p0.shape[1]
    D = 2 * Dw
    per_w = E // NW
    CH = 48
    n_chunks = per_w // CH
    mesh = plsc.VectorSubcoreMesh(core_axis_name="c", subcore_axis_name="s")

    @functools.partial(
        pl.kernel,
        out_type=jax.ShapeDtypeStruct((E, D), jnp.float32),
        mesh=mesh,
        scratch_types=[
            pltpu.VMEM((per_w,), jnp.int32),
            pltpu.VMEM((per_w,), jnp.int32),
            pltpu.VMEM((CH, Dw), jnp.int32),
            pltpu.VMEM((CH, Dw), jnp.int32),
            pltpu.VMEM((CH, D), jnp.float32),
            pltpu.VMEM((CH, Dw), jnp.int32),
            pltpu.VMEM((CH, Dw), jnp.int32),
            pltpu.VMEM((CH, D), jnp.float32),
            pltpu.SemaphoreType.DMA,
            pltpu.SemaphoreType.DMA,
            pltpu.SemaphoreType.DMA,
            pltpu.SemaphoreType.DMA,
            pltpu.SemaphoreType.DMA,
            pltpu.SemaphoreType.DMA,
        ],
    )
    def k(p0_hbm, p1_hbm, i0_hbm, i1_hbm, out_hbm, i0_v, i1_v,
          buf_a0, buf_b0, buf_o0, buf_a1, buf_b1, buf_o1,
          sem_a0, sem_b0, sem_a1, sem_b1, sem_s0, sem_s1):
        wid = lax.axis_index("s") * NC + lax.axis_index("c")
        base = wid * per_w
        pltpu.sync_copy(i0_hbm.at[pl.ds(base, per_w)], i0_v)
        pltpu.sync_copy(i1_hbm.at[pl.ds(base, per_w)], i1_v)

        bufs = ((buf_a0, buf_b0, buf_o0, sem_a0, sem_b0, sem_s0),
                (buf_a1, buf_b1, buf_o1, sem_a1, sem_b1, sem_s1))

        def issue(slot, ci):
            buf_a, buf_b, _, sem_a, sem_b, _ = bufs[slot]
            off = ci * CH
            pltpu.async_copy(p0_hbm.at[i0_v.at[pl.ds(off, CH)]], buf_a, sem_a)
            pltpu.async_copy(p1_hbm.at[i1_v.at[pl.ds(off, CH)]], buf_b, sem_b)

        def wait_gathers(slot, ci):
            buf_a, buf_b, _, sem_a, sem_b, _ = bufs[slot]
            off = ci * CH
            pltpu.make_async_copy(
                p0_hbm.at[i0_v.at[pl.ds(off, CH)]], buf_a, sem_a).wait()
            pltpu.make_async_copy(
                p1_hbm.at[i1_v.at[pl.ds(off, CH)]], buf_b, sem_b).wait()

        def combine(slot):
            buf_a, buf_b, buf_o, _, _, _ = bufs[slot]
            bc = lax.bitcast_convert_type

            @plsc.parallel_loop(0, CH, 1, unroll=4)
            def _(e):
                for j in range(Dw // L):
                    sl = pl.ds(j * L, L)
                    a = buf_a[e, sl]
                    b = buf_b[e, sl]
                    lo = (bc(lax.shift_left(a, 16), jnp.float32)
                          + bc(lax.shift_left(b, 16), jnp.float32))
                    hi = (bc(a & jnp.int32(-65536), jnp.float32)
                          + bc(b & jnp.int32(-65536), jnp.float32))
                    buf_o[e, pl.ds(j * L, L)] = lo
                    buf_o[e, pl.ds(Dw + j * L, L)] = hi

        def issue_store(slot, ci):
            _, _, buf_o, _, _, sem_s = bufs[slot]
            pltpu.async_copy(buf_o, out_hbm.at[pl.ds(base + ci * CH, CH)],
                             sem_s)

        def wait_store(slot, ci):
            _, _, buf_o, _, _, sem_s = bufs[slot]
            pltpu.make_async_copy(
                buf_o, out_hbm.at[pl.ds(base + ci * CH, CH)], sem_s).wait()

        n_pairs = (n_chunks - 1) // 2
        issue(0, 0)
        issue(1, 1)

        def pair(kk, carry):
            c0 = 2 * kk
            wait_gathers(0, c0)
            combine(0)
            issue_store(0, c0)
            wait_gathers(1, c0 + 1)
            combine(1)
            issue_store(1, c0 + 1)

            @pl.when(kk + 1 < n_pairs)
            def _():
                wait_store(0, c0)
                issue(0, c0 + 2)
                wait_store(1, c0 + 1)
                issue(1, c0 + 3)

            return carry

        lax.fori_loop(0, n_pairs, pair, 0, unroll=False)
        last = n_chunks - 1
        wait_store(0, last - 2)
        wait_store(1, last - 1)
        issue(0, last)
        wait_gathers(0, last)
        combine(0)
        pltpu.sync_copy(buf_o0, out_hbm.at[pl.ds(base + last * CH, CH)])

    return k(p0, p1, idx0, idx1)


def _edge_stats(g, nbr, w2t_s):
    E, Bf = nbr.shape
    D = w2t_s.shape[1]
    BE = 2000

    def body(g_ref, nbr_ref, w2_ref, sum_ref, sq_ref):
        i = pl.program_id(0)
        gated = g_ref[...] + jnp.dot(
            nbr_ref[...], w2_ref[...], preferred_element_type=jnp.float32)
        s = jnp.sum(gated, axis=0, keepdims=True)
        q = jnp.sum(gated * gated, axis=0, keepdims=True)

        @pl.when(i == 0)
        def _():
            sum_ref[...] = s
            sq_ref[...] = q

        @pl.when(i != 0)
        def _():
            sum_ref[...] += s
            sq_ref[...] += q

    return pl.pallas_call(
        body,
        grid=(E // BE,),
        in_specs=[
            pl.BlockSpec((BE, D), lambda i: (i, 0)),
            pl.BlockSpec((BE, Bf), lambda i: (i, 0)),
            pl.BlockSpec((Bf, D), lambda i: (0, 0)),
        ],
        out_specs=[
            pl.BlockSpec((1, D), lambda i: (0, 0)),
            pl.BlockSpec((1, D), lambda i: (0, 0)),
        ],
        out_shape=[jax.ShapeDtypeStruct((1, D), jnp.float32)] * 2,
    )(g, nbr, w2t_s)


def _edge_messages(g, nbr, w2t_s, gsum, gsq, gamma1_s, beta1_s):
    E, Bf = nbr.shape
    D = w2t_s.shape[1]
    Dw = D // 2
    A = D // 2
    H = A // 2
    BE = 2000
    inv_e = 1.0 / E

    def body(g_ref, nbr_ref, w2_ref, sum_ref, sq_ref, gam_ref, bet_ref, o_ref):
        gated = g_ref[...] + jnp.dot(
            nbr_ref[...], w2_ref[...], preferred_element_type=jnp.float32)
        mean = sum_ref[...] * inv_e
        var = sq_ref[...] * inv_e - mean * mean
        scale = lax.rsqrt(var + BN1_EPS) * gam_ref[...]
        shift = bet_ref[...] - mean * scale
        xh = gated * scale + shift
        f = xh[:, :A]
        c = xh[:, A:]
        sig = 1.0 / (1.0 + jnp.exp(-f))
        sp = jnp.maximum(c, 0.0) + jnp.log(1.0 + jnp.exp(-jnp.abs(c)))
        msg = sig * sp
        o_ref[0, ...] = msg[:, :H]
        o_ref[1, ...] = msg[:, H:]

    return pl.pallas_call(
        body,
        grid=(E // BE,),
        in_specs=[
            pl.BlockSpec((BE, D), lambda i: (i, 0)),
            pl.BlockSpec((BE, Bf), lambda i: (i, 0)),
            pl.BlockSpec((Bf, D), lambda i: (0, 0)),
            pl.BlockSpec((1, D), lambda i: (0, 0)),
            pl.BlockSpec((1, D), lambda i: (0, 0)),
            pl.BlockSpec((1, D), lambda i: (0, 0)),
            pl.BlockSpec((1, D), lambda i: (0, 0)),
        ],
        out_specs=pl.BlockSpec((2, BE, H), lambda i: (0, i, 0)),
        out_shape=jax.ShapeDtypeStruct((2, E, H), jnp.float32),
    )(g, nbr, w2t_s, gsum, gsq, gamma1_s, beta1_s)


def _sc_scatter_add(msg2, idx_tiled, n_nodes, zeros_init):
    _, E, half = msg2.shape
    per_t = E // NS
    CH = 80
    n_chunks = per_t // CH
    out_writers = 10
    rows_out = n_nodes // out_writers
    mesh = plsc.VectorSubcoreMesh(core_axis_name="c", subcore_axis_name="s")

    @functools.partial(
        pl.kernel,
        out_type=jax.ShapeDtypeStruct((NC, n_nodes, half), jnp.float32),
        mesh=mesh,
        scratch_types=[
            pltpu.VMEM((n_chunks, CH), jnp.int32),
            pltpu.VMEM((CH, half), jnp.float32),
            pltpu.VMEM((CH, half), jnp.float32),
            pltpu.VMEM_SHARED((n_nodes, half), jnp.float32),
            pltpu.SemaphoreType.DMA,
            pltpu.SemaphoreType.DMA,
        ],
    )
    def k(msg_hbm, idx_hbm, zero_hbm, out_hbm, idx_v, buf0, buf1, acc_sh,
          sem0, sem1):
        cid = lax.axis_index("c")
        sid = lax.axis_index("s")
        tbase = sid * per_t
        pltpu.sync_copy(idx_hbm.at[sid], idx_v)

        @pl.when(sid == 0)
        def _():
            pltpu.sync_copy(zero_hbm, acc_sh)

        plsc.subcore_barrier()

        bufs = ((buf0, sem0), (buf1, sem1))

        def issue(slot, j):
            buf, sem = bufs[slot]
            pltpu.async_copy(
                msg_hbm.at[cid, pl.ds(tbase + j * CH, CH)], buf, sem)

        def scat(slot, j):
            buf, sem = bufs[slot]
            pltpu.make_async_copy(
                msg_hbm.at[cid, pl.ds(tbase + j * CH, CH)], buf, sem).wait()
            pltpu.sync_copy(buf, acc_sh.at[idx_v.at[j]], add=True)

        n_pairs = (n_chunks - 1) // 2
        issue(0, 0)
        issue(1, 1)

        def pair(kk, carry):
            j0 = 2 * kk
            scat(0, j0)
            issue(0, j0 + 2)
            scat(1, j0 + 1)

            @pl.when(kk + 1 < n_pairs)
            def _():
                issue(1, j0 + 3)

            return carry

        lax.fori_loop(0, n_pairs, pair, 0, unroll=False)
        scat(0, n_chunks - 1)
        plsc.subcore_barrier()

        @pl.when(sid < out_writers)
        def _():
            pltpu.sync_copy(
                acc_sh.at[pl.ds(sid * rows_out, rows_out)],
                out_hbm.at[cid, pl.ds(sid * rows_out, rows_out)])

    return k(msg2, idx_tiled, zeros_init)


def _finalize(nbr_sumed, atom, gamma2, beta2):
    N, A = atom.shape
    inv_n = 1.0 / N

    def body(s_ref, a_ref, g_ref, b_ref, o_ref):
        x = jnp.concatenate((s_ref[0, ...], s_ref[1, ...]), axis=1)
        mean = jnp.sum(x, axis=0, keepdims=True) * inv_n
        d = x - mean
        var = jnp.sum(d * d, axis=0, keepdims=True) * inv_n
        xh = d * lax.rsqrt(var + BN2_EPS) * g_ref[...] + b_ref[...]
        y = a_ref[...] + xh
        o_ref[...] = jnp.maximum(y, 0.0) + jnp.log(1.0 + jnp.exp(-jnp.abs(y)))

    return pl.pallas_call(
        body,
        out_shape=jax.ShapeDtypeStruct((N, A), jnp.float32),
    )(nbr_sumed, atom, gamma2, beta2)


def kernel(atom_in_fea, nbr_fea, nbr_fea_idx, W, b, gamma1, beta1, gamma2,
           beta2):
    N, A = atom_in_fea.shape
    E = nbr_fea_idx.shape[0]
    D = 2 * A

    w0t = W[:, :A].T
    w1t = W[:, A:2 * A].T
    w2t = W[:, 2 * A:].T
    idx0 = nbr_fea_idx[:, 0].astype(jnp.int32)
    idx1 = nbr_fea_idx[:, 1].astype(jnp.int32)

    e_pad = -(-E // (NW * 48)) * (NW * 48)
    pad = e_pad - E
    idx0_p = jnp.concatenate([idx0, jnp.zeros((pad,), jnp.int32)])
    idx1_p = jnp.concatenate([idx1, jnp.zeros((pad,), jnp.int32)])

    perm = jnp.arange(D).reshape(2, A).T.reshape(D)
    p0, p1 = _project(atom_in_fea, w0t[:, perm], w1t[:, perm])
    p0i = lax.bitcast_convert_type(p0.reshape(N, A, 2), jnp.int32)
    p1i = lax.bitcast_convert_type(p1.reshape(N, A, 2), jnp.int32)
    g = _sc_gather_combine(p0i, p1i, idx0_p, idx1_p)

    gsum, gsq = _edge_stats(g, nbr_fea, w2t)
    msg = _edge_messages(g, nbr_fea, w2t, gsum, gsq,
                         gamma1.reshape(1, D), beta1.reshape(1, D))

    idx_tiled = idx0.reshape(NS, (E // NS) // 80, 80)
    zeros_init = jnp.zeros((N, A // NC), jnp.float32)
    nbr_sumed2 = _sc_scatter_add(msg, idx_tiled, N, zeros_init)

    return _finalize(nbr_sumed2, atom_in_fea,
                     gamma2.reshape(1, A), beta2.reshape(1, A))


# speedup vs baseline: 1.5211x; 1.2229x over previous
import functools

import jax
import jax.numpy as jnp
from jax import lax
from jax.experimental import pallas as pl
from jax.experimental.pallas import tpu as pltpu
from jax.experimental.pallas import tpu_sc as plsc

NC, NS, L = 2, 16, 16
NW = NC * NS
BN1_EPS = 1e-5
BN2_EPS = 1e-5


def _project(atom, w0t, w1t):
    N, A = atom.shape
    D = w0t.shape[1]
    BN = 2000

    def body(x_ref, w0_ref, w1_ref, p0_ref, p1_ref):
        x = x_ref[...]
        p0_ref[...] = jnp.dot(x, w0_ref[...],
                              preferred_element_type=jnp.float32)
        p1_ref[...] = jnp.dot(x, w1_ref[...],
                              preferred_element_type=jnp.float32)

    return pl.pallas_call(
        body,
        grid=(N // BN,),
        in_specs=[
            pl.BlockSpec((BN, A), lambda i: (i, 0)),
            pl.BlockSpec((A, D), lambda i: (0, 0)),
            pl.BlockSpec((A, D), lambda i: (0, 0)),
        ],
        out_specs=[
            pl.BlockSpec((BN, D), lambda i: (i, 0)),
            pl.BlockSpec((BN, D), lambda i: (i, 0)),
        ],
        out_shape=[jax.ShapeDtypeStruct((N, D), jnp.float32)] * 2,
    )(atom, w0t, w1t)


def _sc_gather_combine(p0, p1, idx0, idx1):
    E = idx0.shape[0]
    D = p0.shape[1]
    per_w = E // NW
    CH = 48
    n_chunks = per_w // CH
    mesh = plsc.VectorSubcoreMesh(core_axis_name="c", subcore_axis_name="s")

    @functools.partial(
        pl.kernel,
        out_type=jax.ShapeDtypeStruct((E, D), jnp.float32),
        mesh=mesh,
        scratch_types=[
            pltpu.VMEM((per_w,), jnp.int32),
            pltpu.VMEM((per_w,), jnp.int32),
            pltpu.VMEM((CH, D), jnp.float32),
            pltpu.VMEM((CH, D), jnp.float32),
            pltpu.VMEM((CH, D), jnp.float32),
            pltpu.VMEM((CH, D), jnp.float32),
            pltpu.SemaphoreType.DMA,
            pltpu.SemaphoreType.DMA,
            pltpu.SemaphoreType.DMA,
            pltpu.SemaphoreType.DMA,
            pltpu.SemaphoreType.DMA,
            pltpu.SemaphoreType.DMA,
        ],
    )
    def k(p0_hbm, p1_hbm, i0_hbm, i1_hbm, out_hbm, i0_v, i1_v,
          buf_a0, buf_b0, buf_a1, buf_b1,
          sem_a0, sem_b0, sem_a1, sem_b1, sem_s0, sem_s1):
        wid = lax.axis_index("s") * NC + lax.axis_index("c")
        base = wid * per_w
        pltpu.sync_copy(i0_hbm.at[pl.ds(base, per_w)], i0_v)
        pltpu.sync_copy(i1_hbm.at[pl.ds(base, per_w)], i1_v)

        bufs = ((buf_a0, buf_b0, sem_a0, sem_b0, sem_s0),
                (buf_a1, buf_b1, sem_a1, sem_b1, sem_s1))

        def issue(slot, ci):
            buf_a, buf_b, sem_a, sem_b, _ = bufs[slot]
            off = ci * CH
            pltpu.async_copy(p0_hbm.at[i0_v.at[pl.ds(off, CH)]], buf_a, sem_a)
            pltpu.async_copy(p1_hbm.at[i1_v.at[pl.ds(off, CH)]], buf_b, sem_b)

        def wait_gathers(slot, ci):
            buf_a, buf_b, sem_a, sem_b, _ = bufs[slot]
            off = ci * CH
            pltpu.make_async_copy(
                p0_hbm.at[i0_v.at[pl.ds(off, CH)]], buf_a, sem_a).wait()
            pltpu.make_async_copy(
                p1_hbm.at[i1_v.at[pl.ds(off, CH)]], buf_b, sem_b).wait()

        def combine(slot):
            buf_a, buf_b, _, _, _ = bufs[slot]

            @plsc.parallel_loop(0, CH, 1, unroll=4)
            def _(e):
                for j in range(D // L):
                    sl = pl.ds(j * L, L)
                    plsc.addupdate(buf_a.at[e, sl], buf_b[e, sl])

        def issue_store(slot, ci):
            buf_a, _, _, _, sem_s = bufs[slot]
            pltpu.async_copy(buf_a, out_hbm.at[pl.ds(base + ci * CH, CH)],
                             sem_s)

        def wait_store(slot, ci):
            buf_a, _, _, _, sem_s = bufs[slot]
            pltpu.make_async_copy(
                buf_a, out_hbm.at[pl.ds(base + ci * CH, CH)], sem_s).wait()

        n_pairs = (n_chunks - 1) // 2
        issue(0, 0)
        issue(1, 1)

        def pair(kk, carry):
            c0 = 2 * kk
            wait_gathers(0, c0)
            combine(0)
            issue_store(0, c0)
            wait_gathers(1, c0 + 1)
            combine(1)
            issue_store(1, c0 + 1)

            @pl.when(kk + 1 < n_pairs)
            def _():
                wait_store(0, c0)
                issue(0, c0 + 2)
                wait_store(1, c0 + 1)
                issue(1, c0 + 3)

            return carry

        lax.fori_loop(0, n_pairs, pair, 0, unroll=False)
        last = n_chunks - 1
        wait_store(0, last - 2)
        wait_store(1, last - 1)
        issue(0, last)
        wait_gathers(0, last)
        combine(0)
        pltpu.sync_copy(buf_a0, out_hbm.at[pl.ds(base + last * CH, CH)])

    return k(p0, p1, idx0, idx1)


def _edge_stats(g, nbr, w2t_s):
    E, Bf = nbr.shape
    D = w2t_s.shape[1]
    BE = 2000

    def body(g_ref, nbr_ref, w2_ref, sum_ref, sq_ref):
        i = pl.program_id(0)
        gated = g_ref[...] + jnp.dot(
            nbr_ref[...], w2_ref[...], preferred_element_type=jnp.float32)
        s = jnp.sum(gated, axis=0, keepdims=True)
        q = jnp.sum(gated * gated, axis=0, keepdims=True)

        @pl.when(i == 0)
        def _():
            sum_ref[...] = s
            sq_ref[...] = q

        @pl.when(i != 0)
        def _():
            sum_ref[...] += s
            sq_ref[...] += q

    return pl.pallas_call(
        body,
        grid=(E // BE,),
        in_specs=[
            pl.BlockSpec((BE, D), lambda i: (i, 0)),
            pl.BlockSpec((BE, Bf), lambda i: (i, 0)),
            pl.BlockSpec((Bf, D), lambda i: (0, 0)),
        ],
        out_specs=[
            pl.BlockSpec((1, D), lambda i: (0, 0)),
            pl.BlockSpec((1, D), lambda i: (0, 0)),
        ],
        out_shape=[jax.ShapeDtypeStruct((1, D), jnp.float32)] * 2,
    )(g, nbr, w2t_s)


def _edge_messages(g, nbr, w2t_s, gsum, gsq, gamma1_s, beta1_s):
    E, Bf = nbr.shape
    D = w2t_s.shape[1]
    Dw = D // 2
    A = D // 2
    H = A // 2
    BE = 2000
    inv_e = 1.0 / E

    def body(g_ref, nbr_ref, w2_ref, sum_ref, sq_ref, gam_ref, bet_ref, o_ref):
        gated = g_ref[...] + jnp.dot(
            nbr_ref[...], w2_ref[...], preferred_element_type=jnp.float32)
        mean = sum_ref[...] * inv_e
        var = sq_ref[...] * inv_e - mean * mean
        scale = lax.rsqrt(var + BN1_EPS) * gam_ref[...]
        shift = bet_ref[...] - mean * scale
        xh = gated * scale + shift
        f = xh[:, :A]
        c = xh[:, A:]
        sig = 1.0 / (1.0 + jnp.exp(-f))
        sp = jnp.maximum(c, 0.0) + jnp.log(1.0 + jnp.exp(-jnp.abs(c)))
        msg = sig * sp
        o_ref[0, ...] = msg[:, :H]
        o_ref[1, ...] = msg[:, H:]

    return pl.pallas_call(
        body,
        grid=(E // BE,),
        in_specs=[
            pl.BlockSpec((BE, D), lambda i: (i, 0)),
            pl.BlockSpec((BE, Bf), lambda i: (i, 0)),
            pl.BlockSpec((Bf, D), lambda i: (0, 0)),
            pl.BlockSpec((1, D), lambda i: (0, 0)),
            pl.BlockSpec((1, D), lambda i: (0, 0)),
            pl.BlockSpec((1, D), lambda i: (0, 0)),
            pl.BlockSpec((1, D), lambda i: (0, 0)),
        ],
        out_specs=pl.BlockSpec((2, BE, H), lambda i: (0, i, 0)),
        out_shape=jax.ShapeDtypeStruct((2, E, H), jnp.float32),
    )(g, nbr, w2t_s, gsum, gsq, gamma1_s, beta1_s)


def _sc_scatter_add(msg2, idx_tiled, n_nodes, zeros_init):
    _, E, half = msg2.shape
    per_t = E // NS
    CH = 80
    n_chunks = per_t // CH
    out_writers = 10
    rows_out = n_nodes // out_writers
    mesh = plsc.VectorSubcoreMesh(core_axis_name="c", subcore_axis_name="s")

    @functools.partial(
        pl.kernel,
        out_type=jax.ShapeDtypeStruct((NC, n_nodes, half), jnp.float32),
        mesh=mesh,
        scratch_types=[
            pltpu.VMEM((n_chunks, CH), jnp.int32),
            pltpu.VMEM((CH, half), jnp.float32),
            pltpu.VMEM((CH, half), jnp.float32),
            pltpu.VMEM_SHARED((n_nodes, half), jnp.float32),
            pltpu.SemaphoreType.DMA,
            pltpu.SemaphoreType.DMA,
        ],
    )
    def k(msg_hbm, idx_hbm, zero_hbm, out_hbm, idx_v, buf0, buf1, acc_sh,
          sem0, sem1):
        cid = lax.axis_index("c")
        sid = lax.axis_index("s")
        tbase = sid * per_t
        pltpu.sync_copy(idx_hbm.at[sid], idx_v)

        @pl.when(sid == 0)
        def _():
            pltpu.sync_copy(zero_hbm, acc_sh)

        plsc.subcore_barrier()

        bufs = ((buf0, sem0), (buf1, sem1))

        def issue(slot, j):
            buf, sem = bufs[slot]
            pltpu.async_copy(
                msg_hbm.at[cid, pl.ds(tbase + j * CH, CH)], buf, sem)

        def scat(slot, j):
            buf, sem = bufs[slot]
            pltpu.make_async_copy(
                msg_hbm.at[cid, pl.ds(tbase + j * CH, CH)], buf, sem).wait()
            pltpu.sync_copy(buf, acc_sh.at[idx_v.at[j]], add=True)

        n_pairs = (n_chunks - 1) // 2
        issue(0, 0)
        issue(1, 1)

        def pair(kk, carry):
            j0 = 2 * kk
            scat(0, j0)
            issue(0, j0 + 2)
            scat(1, j0 + 1)

            @pl.when(kk + 1 < n_pairs)
            def _():
                issue(1, j0 + 3)

            return carry

        lax.fori_loop(0, n_pairs, pair, 0, unroll=False)
        scat(0, n_chunks - 1)
        plsc.subcore_barrier()

        @pl.when(sid < out_writers)
        def _():
            pltpu.sync_copy(
                acc_sh.at[pl.ds(sid * rows_out, rows_out)],
                out_hbm.at[cid, pl.ds(sid * rows_out, rows_out)])

    return k(msg2, idx_tiled, zeros_init)


def _finalize(nbr_sumed, atom, gamma2, beta2):
    N, A = atom.shape
    inv_n = 1.0 / N

    def body(s_ref, a_ref, g_ref, b_ref, o_ref):
        x = jnp.concatenate((s_ref[0, ...], s_ref[1, ...]), axis=1)
        mean = jnp.sum(x, axis=0, keepdims=True) * inv_n
        d = x - mean
        var = jnp.sum(d * d, axis=0, keepdims=True) * inv_n
        xh = d * lax.rsqrt(var + BN2_EPS) * g_ref[...] + b_ref[...]
        y = a_ref[...] + xh
        o_ref[...] = jnp.maximum(y, 0.0) + jnp.log(1.0 + jnp.exp(-jnp.abs(y)))

    return pl.pallas_call(
        body,
        out_shape=jax.ShapeDtypeStruct((N, A), jnp.float32),
    )(nbr_sumed, atom, gamma2, beta2)


def kernel(atom_in_fea, nbr_fea, nbr_fea_idx, W, b, gamma1, beta1, gamma2,
           beta2):
    N, A = atom_in_fea.shape
    E = nbr_fea_idx.shape[0]
    D = 2 * A

    w0t = W[:, :A].T
    w1t = W[:, A:2 * A].T
    w2t = W[:, 2 * A:].T
    idx0 = nbr_fea_idx[:, 0].astype(jnp.int32)
    idx1 = nbr_fea_idx[:, 1].astype(jnp.int32)

    e_pad = -(-E // (NW * 48)) * (NW * 48)
    pad = e_pad - E
    idx0_p = jnp.concatenate([idx0, jnp.zeros((pad,), jnp.int32)])
    idx1_p = jnp.concatenate([idx1, jnp.zeros((pad,), jnp.int32)])

    p0, p1 = _project(atom_in_fea, w0t, w1t)
    g = _sc_gather_combine(p0, p1, idx0_p, idx1_p)

    gsum, gsq = _edge_stats(g, nbr_fea, w2t)
    msg = _edge_messages(g, nbr_fea, w2t, gsum, gsq,
                         gamma1.reshape(1, D), beta1.reshape(1, D))

    idx_tiled = idx0.reshape(NS, (E // NS) // 80, 80)
    zeros_init = jnp.zeros((N, A // NC), jnp.float32)
    nbr_sumed2 = _sc_scatter_add(msg, idx_tiled, N, zeros_init)

    return _finalize(nbr_sumed2, atom_in_fea,
                     gamma2.reshape(1, A), beta2.reshape(1, A))


# speedup vs baseline: 1.6948x; 1.1142x over previous
import functools

import jax
import jax.numpy as jnp
from jax import lax
from jax.experimental import pallas as pl
from jax.experimental.pallas import tpu as pltpu
from jax.experimental.pallas import tpu_sc as plsc

NC, NS, L = 2, 16, 16
NW = NC * NS
BN1_EPS = 1e-5
BN2_EPS = 1e-5


def _project(atom, w0t, w1t):
    N, A = atom.shape
    D = w0t.shape[1]
    BN = 2000

    def body(x_ref, w0_ref, w1_ref, p0_ref, p1_ref):
        x = x_ref[...]
        p0_ref[...] = jnp.dot(x, w0_ref[...],
                              preferred_element_type=jnp.float32)
        p1_ref[...] = jnp.dot(x, w1_ref[...],
                              preferred_element_type=jnp.float32)

    return pl.pallas_call(
        body,
        grid=(N // BN,),
        in_specs=[
            pl.BlockSpec((BN, A), lambda i: (i, 0)),
            pl.BlockSpec((A, D), lambda i: (0, 0)),
            pl.BlockSpec((A, D), lambda i: (0, 0)),
        ],
        out_specs=[
            pl.BlockSpec((BN, D), lambda i: (i, 0)),
            pl.BlockSpec((BN, D), lambda i: (i, 0)),
        ],
        out_shape=[jax.ShapeDtypeStruct((N, D), jnp.float32)] * 2,
    )(atom, w0t, w1t)


def _sc_gather_combine(p0, p1, idx0, idx1):
    E = idx0.shape[0]
    D = p0.shape[1]
    per_w = E // NW
    CH = 40
    n_chunks = per_w // CH
    mesh = plsc.VectorSubcoreMesh(core_axis_name="c", subcore_axis_name="s")

    @functools.partial(
        pl.kernel,
        out_type=jax.ShapeDtypeStruct((E, D), jnp.float32),
        mesh=mesh,
        scratch_types=[
            pltpu.VMEM((per_w,), jnp.int32),
            pltpu.VMEM((per_w,), jnp.int32),
            pltpu.VMEM((CH, D), jnp.float32),
            pltpu.VMEM((CH, D), jnp.float32),
            pltpu.VMEM((CH, D), jnp.float32),
            pltpu.VMEM((CH, D), jnp.float32),
            pltpu.SemaphoreType.DMA,
            pltpu.SemaphoreType.DMA,
            pltpu.SemaphoreType.DMA,
            pltpu.SemaphoreType.DMA,
            pltpu.SemaphoreType.DMA,
            pltpu.SemaphoreType.DMA,
        ],
    )
    def k(p0_hbm, p1_hbm, i0_hbm, i1_hbm, out_hbm, i0_v, i1_v,
          buf_a0, buf_b0, buf_a1, buf_b1,
          sem_a0, sem_b0, sem_a1, sem_b1, sem_s0, sem_s1):
        wid = lax.axis_index("s") * NC + lax.axis_index("c")
        base = wid * per_w
        pltpu.sync_copy(i0_hbm.at[pl.ds(base, per_w)], i0_v)
        pltpu.sync_copy(i1_hbm.at[pl.ds(base, per_w)], i1_v)

        bufs = ((buf_a0, buf_b0, sem_a0, sem_b0, sem_s0),
                (buf_a1, buf_b1, sem_a1, sem_b1, sem_s1))

        def issue(slot, ci):
            buf_a, buf_b, sem_a, sem_b, _ = bufs[slot]
            off = ci * CH
            pltpu.async_copy(p0_hbm.at[i0_v.at[pl.ds(off, CH)]], buf_a, sem_a)
            pltpu.async_copy(p1_hbm.at[i1_v.at[pl.ds(off, CH)]], buf_b, sem_b)

        def wait_gathers(slot, ci):
            buf_a, buf_b, sem_a, sem_b, _ = bufs[slot]
            off = ci * CH
            pltpu.make_async_copy(
                p0_hbm.at[i0_v.at[pl.ds(off, CH)]], buf_a, sem_a).wait()
            pltpu.make_async_copy(
                p1_hbm.at[i1_v.at[pl.ds(off, CH)]], buf_b, sem_b).wait()

        def combine(slot):
            buf_a, buf_b, _, _, _ = bufs[slot]

            def row(e, c2):
                for j in range(D // L):
                    sl = pl.ds(j * L, L)
                    plsc.addupdate(buf_a.at[e, sl], buf_b[e, sl])
                return c2

            lax.fori_loop(0, CH, row, 0, unroll=False)

        def issue_store(slot, ci):
            buf_a, _, _, _, sem_s = bufs[slot]
            pltpu.async_copy(buf_a, out_hbm.at[pl.ds(base + ci * CH, CH)],
                             sem_s)

        def wait_store(slot, ci):
            buf_a, _, _, _, sem_s = bufs[slot]
            pltpu.make_async_copy(
                buf_a, out_hbm.at[pl.ds(base + ci * CH, CH)], sem_s).wait()

        n_pairs = (n_chunks - 1) // 2
        issue(0, 0)
        issue(1, 1)

        def pair(kk, carry):
            c0 = 2 * kk
            wait_gathers(0, c0)
            combine(0)
            issue_store(0, c0)
            wait_gathers(1, c0 + 1)
            combine(1)
            issue_store(1, c0 + 1)

            @pl.when(kk + 1 < n_pairs)
            def _():
                wait_store(0, c0)
                issue(0, c0 + 2)
                wait_store(1, c0 + 1)
                issue(1, c0 + 3)

            return carry

        lax.fori_loop(0, n_pairs, pair, 0, unroll=False)
        last = n_chunks - 1
        wait_store(0, last - 2)
        wait_store(1, last - 1)
        issue(0, last)
        wait_gathers(0, last)
        combine(0)
        pltpu.sync_copy(buf_a0, out_hbm.at[pl.ds(base + last * CH, CH)])

    return k(p0, p1, idx0, idx1)


def _edge_stats(g, nbr, w2t_s):
    E, Bf = nbr.shape
    D = w2t_s.shape[1]
    BE = 2000

    def body(g_ref, nbr_ref, w2_ref, sum_ref, sq_ref):
        i = pl.program_id(0)
        gated = g_ref[...] + jnp.dot(
            nbr_ref[...], w2_ref[...], preferred_element_type=jnp.float32)
        s = jnp.sum(gated, axis=0, keepdims=True)
        q = jnp.sum(gated * gated, axis=0, keepdims=True)

        @pl.when(i == 0)
        def _():
            sum_ref[...] = s
            sq_ref[...] = q

        @pl.when(i != 0)
        def _():
            sum_ref[...] += s
            sq_ref[...] += q

    return pl.pallas_call(
        body,
        grid=(E // BE,),
        in_specs=[
            pl.BlockSpec((BE, D), lambda i: (i, 0)),
            pl.BlockSpec((BE, Bf), lambda i: (i, 0)),
            pl.BlockSpec((Bf, D), lambda i: (0, 0)),
        ],
        out_specs=[
            pl.BlockSpec((1, D), lambda i: (0, 0)),
            pl.BlockSpec((1, D), lambda i: (0, 0)),
        ],
        out_shape=[jax.ShapeDtypeStruct((1, D), jnp.float32)] * 2,
    )(g, nbr, w2t_s)


def _edge_messages(g, nbr, w2t_s, gsum, gsq, gamma1_s, beta1_s):
    E, Bf = nbr.shape
    D = w2t_s.shape[1]
    Dw = D // 2
    A = D // 2
    H = A // 2
    BE = 2000
    inv_e = 1.0 / E

    def body(g_ref, nbr_ref, w2_ref, sum_ref, sq_ref, gam_ref, bet_ref, o_ref):
        gated = g_ref[...] + jnp.dot(
            nbr_ref[...], w2_ref[...], preferred_element_type=jnp.float32)
        mean = sum_ref[...] * inv_e
        var = sq_ref[...] * inv_e - mean * mean
        scale = lax.rsqrt(var + BN1_EPS) * gam_ref[...]
        shift = bet_ref[...] - mean * scale
        xh = gated * scale + shift
        f = xh[:, :A]
        c = xh[:, A:]
        sig = 1.0 / (1.0 + jnp.exp(-f))
        sp = jnp.maximum(c, 0.0) + jnp.log(1.0 + jnp.exp(-jnp.abs(c)))
        msg = sig * sp
        o_ref[0, ...] = msg[:, :H]
        o_ref[1, ...] = msg[:, H:]

    return pl.pallas_call(
        body,
        grid=(E // BE,),
        in_specs=[
            pl.BlockSpec((BE, D), lambda i: (i, 0)),
            pl.BlockSpec((BE, Bf), lambda i: (i, 0)),
            pl.BlockSpec((Bf, D), lambda i: (0, 0)),
            pl.BlockSpec((1, D), lambda i: (0, 0)),
            pl.BlockSpec((1, D), lambda i: (0, 0)),
            pl.BlockSpec((1, D), lambda i: (0, 0)),
            pl.BlockSpec((1, D), lambda i: (0, 0)),
        ],
        out_specs=pl.BlockSpec((2, BE, H), lambda i: (0, i, 0)),
        out_shape=jax.ShapeDtypeStruct((2, E, H), jnp.float32),
    )(g, nbr, w2t_s, gsum, gsq, gamma1_s, beta1_s)


def _sc_scatter_add(msg2, idx_tiled, n_nodes, zeros_init):
    _, E, half = msg2.shape
    per_t = E // NS
    CH = 80
    n_chunks = per_t // CH
    out_writers = 10
    rows_out = n_nodes // out_writers
    mesh = plsc.VectorSubcoreMesh(core_axis_name="c", subcore_axis_name="s")

    @functools.partial(
        pl.kernel,
        out_type=jax.ShapeDtypeStruct((NC, n_nodes, half), jnp.float32),
        mesh=mesh,
        scratch_types=[
            pltpu.VMEM((n_chunks, CH), jnp.int32),
            pltpu.VMEM((CH, half), jnp.float32),
            pltpu.VMEM((CH, half), jnp.float32),
            pltpu.VMEM_SHARED((n_nodes, half), jnp.float32),
            pltpu.SemaphoreType.DMA,
            pltpu.SemaphoreType.DMA,
        ],
    )
    def k(msg_hbm, idx_hbm, zero_hbm, out_hbm, idx_v, buf0, buf1, acc_sh,
          sem0, sem1):
        cid = lax.axis_index("c")
        sid = lax.axis_index("s")
        tbase = sid * per_t
        pltpu.sync_copy(idx_hbm.at[sid], idx_v)

        @pl.when(sid == 0)
        def _():
            pltpu.sync_copy(zero_hbm, acc_sh)

        plsc.subcore_barrier()

        bufs = ((buf0, sem0), (buf1, sem1))

        def issue(slot, j):
            buf, sem = bufs[slot]
            pltpu.async_copy(
                msg_hbm.at[cid, pl.ds(tbase + j * CH, CH)], buf, sem)

        def scat(slot, j):
            buf, sem = bufs[slot]
            pltpu.make_async_copy(
                msg_hbm.at[cid, pl.ds(tbase + j * CH, CH)], buf, sem).wait()
            pltpu.sync_copy(buf, acc_sh.at[idx_v.at[j]], add=True)

        n_pairs = (n_chunks - 1) // 2
        issue(0, 0)
        issue(1, 1)

        def pair(kk, carry):
            j0 = 2 * kk
            scat(0, j0)
            issue(0, j0 + 2)
            scat(1, j0 + 1)

            @pl.when(kk + 1 < n_pairs)
            def _():
                issue(1, j0 + 3)

            return carry

        lax.fori_loop(0, n_pairs, pair, 0, unroll=False)
        scat(0, n_chunks - 1)
        plsc.subcore_barrier()

        @pl.when(sid < out_writers)
        def _():
            pltpu.sync_copy(
                acc_sh.at[pl.ds(sid * rows_out, rows_out)],
                out_hbm.at[cid, pl.ds(sid * rows_out, rows_out)])

    return k(msg2, idx_tiled, zeros_init)


def _finalize(nbr_sumed, atom, gamma2, beta2):
    N, A = atom.shape
    inv_n = 1.0 / N

    def body(s_ref, a_ref, g_ref, b_ref, o_ref):
        x = jnp.concatenate((s_ref[0, ...], s_ref[1, ...]), axis=1)
        mean = jnp.sum(x, axis=0, keepdims=True) * inv_n
        d = x - mean
        var = jnp.sum(d * d, axis=0, keepdims=True) * inv_n
        xh = d * lax.rsqrt(var + BN2_EPS) * g_ref[...] + b_ref[...]
        y = a_ref[...] + xh
        o_ref[...] = jnp.maximum(y, 0.0) + jnp.log(1.0 + jnp.exp(-jnp.abs(y)))

    return pl.pallas_call(
        body,
        out_shape=jax.ShapeDtypeStruct((N, A), jnp.float32),
    )(nbr_sumed, atom, gamma2, beta2)


def kernel(atom_in_fea, nbr_fea, nbr_fea_idx, W, b, gamma1, beta1, gamma2,
           beta2):
    N, A = atom_in_fea.shape
    E = nbr_fea_idx.shape[0]
    D = 2 * A

    w0t = W[:, :A].T
    w1t = W[:, A:2 * A].T
    w2t = W[:, 2 * A:].T
    idx0 = nbr_fea_idx[:, 0].astype(jnp.int32)
    idx1 = nbr_fea_idx[:, 1].astype(jnp.int32)


    p0, p1 = _project(atom_in_fea, w0t, w1t)
    g = _sc_gather_combine(p0, p1, idx0, idx1)

    gsum, gsq = _edge_stats(g, nbr_fea, w2t)
    msg = _edge_messages(g, nbr_fea, w2t, gsum, gsq,
                         gamma1.reshape(1, D), beta1.reshape(1, D))

    idx_tiled = idx0.reshape(NS, (E // NS) // 80, 80)
    zeros_init = jnp.zeros((N, A // NC), jnp.float32)
    nbr_sumed2 = _sc_scatter_add(msg, idx_tiled, N, zeros_init)

    return _finalize(nbr_sumed2, atom_in_fea,
                     gamma2.reshape(1, A), beta2.reshape(1, A))
